# Initial kernel scaffold; baseline (speedup 1.0000x reference)
import jax
import jax.numpy as jnp
from jax.experimental import pallas as pl


def kernel(x, edge_index, W1, att_src1, att_dst1, b1, W2, att_src2, att_dst2, b2):
    raise NotImplementedError("write your pallas kernel here")


# speedup vs baseline: 28.7716x; 28.7716x over previous
import functools
import jax
import jax.numpy as jnp
from jax import lax
from jax.experimental import pallas as pl
from jax.experimental.pallas import tpu as pltpu
from jax.experimental.pallas import tpu_sc as plsc

N = 10000
FEAT = 128
H = 8
HID = 16
NCLS = 64
N_PAD = 10240
E_REAL = 320000 + N
K = 128
NTILES = 32
EPB = K
BLOCKS = -(-E_REAL // (NTILES * K))
E_PAD = NTILES * K * BLOCKS
EPT = E_PAD // NTILES
STRIPE = N_PAD // 16

@functools.cache
def _mesh():
    return plsc.VectorSubcoreMesh(core_axis_name="c", subcore_axis_name="s")
_iota16 = lambda: lax.broadcasted_iota(jnp.int32, (16,), 0)


def _zero_rows(ref, nrows, ncols):
    z = jnp.zeros((16,), jnp.float32)

    def body(i, _):
        for c in range(ncols // 16):
            ref[i, pl.ds(c * 16, 16)] = z
        return 0

    lax.fori_loop(0, nrows, body, 0)


def _pass_a(src, dst, asad):
    def body(src_hbm, dst_hbm, asad_hbm, ex_hbm, denom_hbm,
             denom_sh, src_v, dst_v, gs_v, gd_v, ex_v, zero_v):
        cid = lax.axis_index("c")
        sid = lax.axis_index("s")
        wid = sid * 2 + cid

        _zero_rows(zero_v, K, 16)
        _zero_rows(ex_v, K, 16)
        for r in range(STRIPE // K):
            pltpu.sync_copy(zero_v, denom_sh.at[pl.ds(sid * STRIPE + r * K, K)])
        plsc.subcore_barrier()

        def blk(b, _):
            base = wid * EPT + b * K
            pltpu.sync_copy(src_hbm.at[pl.ds(base, K)], src_v)
            pltpu.sync_copy(dst_hbm.at[pl.ds(base, K)], dst_v)
            pltpu.sync_copy(asad_hbm.at[src_v], gs_v)
            pltpu.sync_copy(asad_hbm.at[dst_v], gd_v)
            it = _iota16()
            for g in range(K // 16):
                row = g * 16 + it
                for h in range(H):
                    a = plsc.load_gather(gs_v, [row, jnp.full((16,), h, jnp.int32)])
                    b2 = plsc.load_gather(gd_v, [row, jnp.full((16,), 8 + h, jnp.int32)])
                    e = a + b2
                    e = jnp.maximum(e, 0.2 * e)
                    x = jnp.exp(e)
                    plsc.store_scatter(ex_v, [row, jnp.full((16,), h, jnp.int32)], x)
            pltpu.sync_copy(ex_v, ex_hbm.at[pl.ds(base, K)])
            pltpu.sync_copy(ex_v, denom_sh.at[dst_v], add=True)
            return 0

        lax.fori_loop(0, BLOCKS, blk, 0)
        plsc.subcore_barrier()
        pltpu.sync_copy(denom_sh.at[pl.ds(sid * STRIPE, STRIPE)],
                        denom_hbm.at[cid, pl.ds(sid * STRIPE, STRIPE)])

    f = pl.kernel(
        body,
        out_type=(jax.ShapeDtypeStruct((E_PAD, 16), jnp.float32),
                  jax.ShapeDtypeStruct((2, N_PAD, 16), jnp.float32)),
        mesh=_mesh(),
        compiler_params=pltpu.CompilerParams(needs_layout_passes=False, use_tc_tiling_on_sc=False),
        scratch_types=[
            pltpu.VMEM_SHARED((N_PAD, 16), jnp.float32),
            pltpu.VMEM((K,), jnp.int32),
            pltpu.VMEM((K,), jnp.int32),
            pltpu.VMEM((K, 16), jnp.float32),
            pltpu.VMEM((K, 16), jnp.float32),
            pltpu.VMEM((K, 16), jnp.float32),
            pltpu.VMEM((K, 16), jnp.float32),
        ],
    )
    return f(src, dst, asad)


def _pass_b1(src, dst, ex, h1):
    def body(src_hbm, dst_hbm, ex_hbm, h_hbm, acc_hbm,
             acc_sh, src_v, dst_v, ex_v, rows_v, msg_v):
        cid = lax.axis_index("c")
        sid = lax.axis_index("s")
        wid = sid * 2 + cid

        _zero_rows(msg_v, K, FEAT)
        for r in range(STRIPE // K):
            pltpu.sync_copy(msg_v, acc_sh.at[pl.ds(sid * STRIPE + r * K, K)])
        plsc.subcore_barrier()

        def blk(b, _):
            base = wid * EPT + b * K
            pltpu.sync_copy(src_hbm.at[pl.ds(base, K)], src_v)
            pltpu.sync_copy(dst_hbm.at[pl.ds(base, K)], dst_v)
            pltpu.sync_copy(ex_hbm.at[pl.ds(base, K)], ex_v)
            pltpu.sync_copy(h_hbm.at[src_v], rows_v)

            def edge(j, _):
                jj = jnp.full((16,), j, jnp.int32)
                for h in range(H):
                    w = plsc.load_gather(ex_v, [jj, jnp.full((16,), h, jnp.int32)])
                    chunk = rows_v[j, pl.ds(h * 16, 16)]
                    msg_v[j, pl.ds(h * 16, 16)] = w * chunk
                return 0

            lax.fori_loop(0, K, edge, 0)
            pltpu.sync_copy(msg_v, acc_sh.at[dst_v], add=True)
            return 0

        lax.fori_loop(0, BLOCKS, blk, 0)
        plsc.subcore_barrier()
        pltpu.sync_copy(acc_sh.at[pl.ds(sid * STRIPE, STRIPE)],
                        acc_hbm.at[cid, pl.ds(sid * STRIPE, STRIPE)])

    f = pl.kernel(
        body,
        out_type=jax.ShapeDtypeStruct((2, N_PAD, FEAT), jnp.float32),
        mesh=_mesh(),
        compiler_params=pltpu.CompilerParams(needs_layout_passes=False, use_tc_tiling_on_sc=False),
        scratch_types=[
            pltpu.VMEM_SHARED((N_PAD, FEAT), jnp.float32),
            pltpu.VMEM((K,), jnp.int32),
            pltpu.VMEM((K,), jnp.int32),
            pltpu.VMEM((K, 16), jnp.float32),
            pltpu.VMEM((K, FEAT), jnp.float32),
            pltpu.VMEM((K, FEAT), jnp.float32),
        ],
    )
    return f(src, dst, ex, h1)


def _pass_b2(src, dst, ex, da, db, h2):
    D = H * NCLS

    def body(src_hbm, dst_hbm, ex_hbm, da_hbm, db_hbm, h_hbm, acc_hbm,
             acc_sh, src_v, dst_v, ex_v, d0_v, d1_v, w_v, rows_v, msg_v):
        cid = lax.axis_index("c")
        sid = lax.axis_index("s")
        wid = sid * 2 + cid

        _zero_rows(msg_v, K, NCLS)
        for r in range(STRIPE // K):
            pltpu.sync_copy(msg_v, acc_sh.at[pl.ds(sid * STRIPE + r * K, K)])
        plsc.subcore_barrier()

        def blk(b, _):
            base = wid * EPT + b * K
            pltpu.sync_copy(src_hbm.at[pl.ds(base, K)], src_v)
            pltpu.sync_copy(dst_hbm.at[pl.ds(base, K)], dst_v)
            pltpu.sync_copy(ex_hbm.at[pl.ds(base, K)], ex_v)
            pltpu.sync_copy(da_hbm.at[dst_v], d0_v)
            pltpu.sync_copy(db_hbm.at[dst_v], d1_v)
            pltpu.sync_copy(h_hbm.at[src_v], rows_v)
            it = _iota16()
            for g in range(K // 16):
                row = g * 16 + it
                for h in range(H):
                    hh = jnp.full((16,), h, jnp.int32)
                    exv = plsc.load_gather(ex_v, [row, hh])
                    dav = plsc.load_gather(d0_v, [row, hh])
                    dbv = plsc.load_gather(d1_v, [row, hh])
                    w = exv / (dav + dbv + 1e-16) * (1.0 / H)
                    plsc.store_scatter(w_v, [row, hh], w)

            def edge(j, _):
                jj = jnp.full((16,), j, jnp.int32)
                ws = [plsc.load_gather(w_v, [jj, jnp.full((16,), h, jnp.int32)])
                      for h in range(H)]
                for cb in range(NCLS // 16):
                    acc = jnp.zeros((16,), jnp.float32)
                    for h in range(H):
                        acc = acc + ws[h] * rows_v[j, pl.ds(h * NCLS + cb * 16, 16)]
                    msg_v[j, pl.ds(cb * 16, 16)] = acc
                return 0

            lax.fori_loop(0, K, edge, 0)
            pltpu.sync_copy(msg_v, acc_sh.at[dst_v], add=True)
            return 0

        lax.fori_loop(0, BLOCKS, blk, 0)
        plsc.subcore_barrier()
        pltpu.sync_copy(acc_sh.at[pl.ds(sid * STRIPE, STRIPE)],
                        acc_hbm.at[cid, pl.ds(sid * STRIPE, STRIPE)])

    f = pl.kernel(
        body,
        out_type=jax.ShapeDtypeStruct((2, N_PAD, NCLS), jnp.float32),
        mesh=_mesh(),
        compiler_params=pltpu.CompilerParams(needs_layout_passes=False, use_tc_tiling_on_sc=False),
        scratch_types=[
            pltpu.VMEM_SHARED((N_PAD, NCLS), jnp.float32),
            pltpu.VMEM((K,), jnp.int32),
            pltpu.VMEM((K,), jnp.int32),
            pltpu.VMEM((K, 16), jnp.float32),
            pltpu.VMEM((K, 16), jnp.float32),
            pltpu.VMEM((K, 16), jnp.float32),
            pltpu.VMEM((K, 16), jnp.float32),
            pltpu.VMEM((K, D), jnp.float32),
            pltpu.VMEM((K, NCLS), jnp.float32),
        ],
    )
    return f(src, dst, ex, da, db, h2)


def _elu(x):
    return jnp.where(x > 0, x, jnp.exp(x) - 1.0)


def _tc_a(x, W1, S1):
    BR = 256

    def body(x_ref, w_ref, s_ref, h_ref, a_ref):
        xe = _elu(x_ref[...])
        h = jnp.dot(xe, w_ref[...], preferred_element_type=jnp.float32)
        h_ref[...] = h
        a_ref[...] = jnp.dot(h, s_ref[...], preferred_element_type=jnp.float32)

    return pl.pallas_call(
        body,
        grid=(N_PAD // BR,),
        in_specs=[
            pl.BlockSpec((BR, FEAT), lambda i: (i, 0)),
            pl.BlockSpec((FEAT, FEAT), lambda i: (0, 0)),
            pl.BlockSpec((FEAT, 16), lambda i: (0, 0)),
        ],
        out_specs=[
            pl.BlockSpec((BR, FEAT), lambda i: (i, 0)),
            pl.BlockSpec((BR, 16), lambda i: (i, 0)),
        ],
        out_shape=[
            jax.ShapeDtypeStruct((N_PAD, FEAT), jnp.float32),
            jax.ShapeDtypeStruct((N_PAD, 16), jnp.float32),
        ],
    )(x, W1, S1)


def _tc_b(a0, a1, d0, d1, b1, W2, S2, R):
    BR = 256
    D = H * NCLS

    def body(a0_ref, a1_ref, d0_ref, d1_ref, b1_ref, w_ref, s_ref, r_ref,
             h_ref, a_ref):
        acc = a0_ref[...] + a1_ref[...]
        den = jnp.dot(d0_ref[...] + d1_ref[...], r_ref[...],
                      preferred_element_type=jnp.float32)
        o1 = acc / (den + 1e-16) + b1_ref[...]
        g = _elu(o1)
        h = jnp.dot(g, w_ref[...], preferred_element_type=jnp.float32)
        h_ref[...] = h
        a_ref[...] = jnp.dot(h, s_ref[...], preferred_element_type=jnp.float32)

    return pl.pallas_call(
        body,
        grid=(N_PAD // BR,),
        in_specs=[
            pl.BlockSpec((BR, FEAT), lambda i: (i, 0)),
            pl.BlockSpec((BR, FEAT), lambda i: (i, 0)),
            pl.BlockSpec((BR, 16), lambda i: (i, 0)),
            pl.BlockSpec((BR, 16), lambda i: (i, 0)),
            pl.BlockSpec((1, FEAT), lambda i: (0, 0)),
            pl.BlockSpec((FEAT, D), lambda i: (0, 0)),
            pl.BlockSpec((D, 16), lambda i: (0, 0)),
            pl.BlockSpec((16, FEAT), lambda i: (0, 0)),
        ],
        out_specs=[
            pl.BlockSpec((BR, D), lambda i: (i, 0)),
            pl.BlockSpec((BR, 16), lambda i: (i, 0)),
        ],
        out_shape=[
            jax.ShapeDtypeStruct((N_PAD, D), jnp.float32),
            jax.ShapeDtypeStruct((N_PAD, 16), jnp.float32),
        ],
    )(a0, a1, d0, d1, b1, W2, S2, R)


def _tc_c(a0, a1, b2):
    BR = 256

    def body(a0_ref, a1_ref, b_ref, o_ref):
        o_ref[...] = a0_ref[...] + a1_ref[...] + b_ref[...]

    return pl.pallas_call(
        body,
        grid=(N_PAD // BR,),
        in_specs=[
            pl.BlockSpec((BR, NCLS), lambda i: (i, 0)),
            pl.BlockSpec((BR, NCLS), lambda i: (i, 0)),
            pl.BlockSpec((1, NCLS), lambda i: (0, 0)),
        ],
        out_specs=pl.BlockSpec((BR, NCLS), lambda i: (i, 0)),
        out_shape=jax.ShapeDtypeStruct((N_PAD, NCLS), jnp.float32),
    )(a0, a1, b2)


@jax.jit
def _run(x, src, dst, W1, S1, b1, W2, S2, R, b2):
    x_pad = jnp.pad(x, ((0, N_PAD - N), (0, 0)))
    h1, asad1 = _tc_a(x_pad, W1, S1)
    ex1, den1 = _pass_a(src, dst, asad1)
    acc1 = _pass_b1(src, dst, ex1, h1)
    h2, asad2 = _tc_b(acc1[0], acc1[1], den1[0], den1[1], b1, W2, S2, R)
    ex2, den2 = _pass_a(src, dst, asad2)
    acc2 = _pass_b2(src, dst, ex2, den2[0], den2[1], h2)
    out = _tc_c(acc2[0], acc2[1], b2)
    return out[:N]


def kernel(x, edge_index, W1, att_src1, att_dst1, b1, W2, att_src2, att_dst2, b2):
    loops = jnp.arange(N, dtype=jnp.int32)
    padi = jnp.full((E_PAD - E_REAL,), N, jnp.int32)
    src = jnp.concatenate([edge_index[0].astype(jnp.int32), loops, padi])
    dst = jnp.concatenate([edge_index[1].astype(jnp.int32), loops, padi])

    m1 = (jnp.arange(FEAT)[:, None] // HID == jnp.arange(H)[None, :]).astype(jnp.float32)
    S1 = jnp.concatenate([m1 * att_src1.reshape(-1)[:, None],
                          m1 * att_dst1.reshape(-1)[:, None]], axis=1)
    D = H * NCLS
    m2 = (jnp.arange(D)[:, None] // NCLS == jnp.arange(H)[None, :]).astype(jnp.float32)
    S2 = jnp.concatenate([m2 * att_src2.reshape(-1)[:, None],
                          m2 * att_dst2.reshape(-1)[:, None]], axis=1)
    R = (jnp.arange(16)[:, None] == jnp.arange(FEAT)[None, :] // HID).astype(jnp.float32)

    return _run(x, src, dst, W1, S1, b1.reshape(1, -1), W2, S2, R,
                b2.reshape(1, -1))
